# baseline (speedup 1.0000x reference)
import functools

import jax
import jax.numpy as jnp
from jax import lax
from jax.experimental import pallas as pl
from jax.experimental.pallas import tpu as pltpu
from jax.experimental.pallas import tpu_sc as plsc

N = 10000
E = 160000
B = 64
EMBED = 256
HALF = 128
T = 4

F32 = jnp.float32


def _dot(a, b, precision=None):
    return lax.dot_general(a, b, (((1,), (0,)), ((), ())),
                           preferred_element_type=F32, precision=precision)


def _node_prologue_body(x_ref, w_ref, p_ref, h_ref, mlo_ref, mhi_ref):
    hb = jax.nn.relu(_dot(x_ref[...], w_ref[...]))
    h_ref[...] = hb
    m = _dot(hb, p_ref[...])
    mlo_ref[...] = m[:, :HALF]
    mhi_ref[...] = m[:, HALF:]


def _node_prologue(x8, w8, p):
    blk = 1000
    return pl.pallas_call(
        _node_prologue_body,
        grid=(N // blk,),
        in_specs=[
            pl.BlockSpec((blk, 8), lambda r: (r, 0)),
            pl.BlockSpec((8, EMBED), lambda r: (0, 0)),
            pl.BlockSpec((EMBED, EMBED), lambda r: (0, 0)),
        ],
        out_specs=[
            pl.BlockSpec((blk, EMBED), lambda r: (r, 0)),
            pl.BlockSpec((blk, HALF), lambda r: (r, 0)),
            pl.BlockSpec((blk, HALF), lambda r: (r, 0)),
        ],
        out_shape=[
            jax.ShapeDtypeStruct((N, EMBED), F32),
            jax.ShapeDtypeStruct((N, HALF), F32),
            jax.ShapeDtypeStruct((N, HALF), F32),
        ],
    )(x8, w8, p)


def _edge_prologue_body(ea_ref, w_ref, lo_ref, hi_ref):
    e = _dot(ea_ref[...], w_ref[...])
    lo_ref[...] = e[:, :HALF]
    hi_ref[...] = e[:, HALF:]


def _edge_prologue(ea8, we8):
    blk = 3456
    rows = ea8.shape[0]
    return pl.pallas_call(
        _edge_prologue_body,
        grid=(rows // blk,),
        in_specs=[
            pl.BlockSpec((blk, 8), lambda r: (r, 0)),
            pl.BlockSpec((8, EMBED), lambda r: (0, 0)),
        ],
        out_specs=[
            pl.BlockSpec((blk, HALF), lambda r: (r, 0)),
            pl.BlockSpec((blk, HALF), lambda r: (r, 0)),
        ],
        out_shape=[
            jax.ShapeDtypeStruct((rows, HALF), F32),
            jax.ShapeDtypeStruct((rows, HALF), F32),
        ],
    )(ea8, we8)


def _update_body(elo_ref, ehi_ref, h_ref, t1lo_ref, t1hi_ref, t2_ref, p_ref,
                 h2_ref, mlo_ref, mhi_ref):
    acc = _dot(elo_ref[...], t1lo_ref[...])
    acc = acc + _dot(ehi_ref[...], t1hi_ref[...])
    acc = acc + _dot(h_ref[...], t2_ref[...])
    hb = jax.nn.relu(acc)
    h2_ref[...] = hb
    m = _dot(hb, p_ref[...])
    mlo_ref[...] = m[:, :HALF]
    mhi_ref[...] = m[:, HALF:]


def _update(e2n_lo, e2n_hi, h, t1_lo, t1_hi, t2, p):
    blk = 1000
    return pl.pallas_call(
        _update_body,
        grid=(N // blk,),
        in_specs=[
            pl.BlockSpec((blk, HALF), lambda r: (r, 0)),
            pl.BlockSpec((blk, HALF), lambda r: (r, 0)),
            pl.BlockSpec((blk, EMBED), lambda r: (r, 0)),
            pl.BlockSpec((HALF, EMBED), lambda r: (0, 0)),
            pl.BlockSpec((HALF, EMBED), lambda r: (0, 0)),
            pl.BlockSpec((EMBED, EMBED), lambda r: (0, 0)),
            pl.BlockSpec((EMBED, EMBED), lambda r: (0, 0)),
        ],
        out_specs=[
            pl.BlockSpec((blk, EMBED), lambda r: (r, 0)),
            pl.BlockSpec((blk, HALF), lambda r: (r, 0)),
            pl.BlockSpec((blk, HALF), lambda r: (r, 0)),
        ],
        out_shape=[
            jax.ShapeDtypeStruct((N, EMBED), F32),
            jax.ShapeDtypeStruct((N, HALF), F32),
            jax.ShapeDtypeStruct((N, HALF), F32),
        ],
    )(e2n_lo, e2n_hi, h, t1_lo, t1_hi, t2, p)


def _epilogue_body(h_ref, batch_ref, y_ref, h1t_ref, h1b_ref, h2_ref, o_ref):
    hv = h_ref[...]
    bsel = jnp.broadcast_to(batch_ref[...], (B, N)) == lax.broadcasted_iota(
        jnp.int32, (B, N), 0)
    bo = jnp.where(bsel, 1.0, 0.0).astype(F32)
    y_pot = _dot(bo, hv, precision=lax.Precision.HIGHEST)
    ysel = lax.broadcasted_iota(jnp.int32, (B, N), 1) == jnp.broadcast_to(
        y_ref[...], (B, N))
    yo = jnp.where(ysel, 1.0, 0.0).astype(F32)
    act = _dot(yo, hv, precision=lax.Precision.HIGHEST)
    hid = jax.nn.relu(_dot(act, h1t_ref[...]) + _dot(y_pot, h1b_ref[...]))
    o_ref[...] = _dot(hid, h2_ref[...])


def _epilogue(h, batch2d, y2d, h1_top, h1_bot, h2):
    return pl.pallas_call(
        _epilogue_body,
        out_shape=jax.ShapeDtypeStruct((B, 1), F32),
    )(h, batch2d, y2d, h1_top, h1_bot, h2)


_CHUNK = 48
_PER_TILE = 216
_NSUB = 16
_CHUNKS = _PER_TILE * _NSUB
E_PAD = _CHUNKS * _CHUNK
N_PAD = 10112
_ROWS_PER_TILE = N_PAD // _NSUB


def _sc_core(msg, ea, out, pk1d, acc, pk, s_st, d_st, ab, bb, gs, es, ss, sid):
    base = sid * _PER_TILE
    nidx = _PER_TILE * _CHUNK

    pltpu.sync_copy(pk1d.at[pl.ds(sid * nidx, nidx)], pk)

    def stage(j, t):
        for k in range(_CHUNK // 16):
            pv = pk[pl.ds(j * _CHUNK + k * 16, 16)]
            s_st[t][pl.ds(k * 16, 16)] = pv & 0xFFFF
            d_st[t][pl.ds(k * 16, 16)] = lax.shift_right_logical(pv, 16)

    def issue_in(j, t):
        pltpu.async_copy(msg.at[s_st[t]], ab[t], gs[t])
        pltpu.async_copy(ea.at[pl.ds((base + j) * _CHUNK, _CHUNK)], bb[t], es[t])

    def wait_in(j, t):
        pltpu.make_async_copy(msg.at[s_st[t]], ab[t], gs[t]).wait()
        pltpu.make_async_copy(ea.at[pl.ds((base + j) * _CHUNK, _CHUNK)], bb[t], es[t]).wait()

    def wait_scatter(t):
        pltpu.make_async_copy(ab[t], acc.at[d_st[t]], ss[t]).wait()

    def compute(t):
        def _rows(r, _):
            for k in range(HALF // 16):
                sl = pl.ds(k * 16, 16)
                ab[t][r, sl] = jnp.maximum(ab[t][r, sl] + bb[t][r, sl], 0.0)
            return 0
        lax.fori_loop(0, _CHUNK, _rows, 0)

    stage(0, 0)
    issue_in(0, 0)
    stage(1, 1)
    issue_in(1, 1)

    def group(g, _):
        for t in range(3):
            j = 3 * g + t
            wait_in(j, t)
            compute(t)
            pltpu.async_copy(ab[t], acc.at[d_st[t]], ss[t], add=True)

            @pl.when(j + 2 < _PER_TILE)
            def _refill():
                t2 = (t + 2) % 3
                if t == 0:
                    @pl.when(g >= 1)
                    def _w():
                        wait_scatter(t2)
                else:
                    wait_scatter(t2)
                stage(j + 2, t2)
                issue_in(j + 2, t2)
        return 0

    lax.fori_loop(0, _PER_TILE // 3, group, 0)
    wait_scatter(0)
    wait_scatter(1)
    wait_scatter(2)

    plsc.subcore_barrier()
    rows = pl.ds(sid * _ROWS_PER_TILE, _ROWS_PER_TILE)
    pltpu.sync_copy(acc.at[rows], out.at[rows])


def _sc_edge_body(mlo, mhi, elo, ehi, pk1d, out_lo, out_hi,
                  pk, s0, s1, s2, d0, d1, d2, a0, a1, a2, b0, b1, b2, acc,
                  g0, g1, g2, e0, e1, e2, sc0, sc1, sc2):
    cid = lax.axis_index("c")
    sid = lax.axis_index("s")

    def zrow(r, _):
        for k in range(HALF // 16):
            a0[r, pl.ds(k * 16, 16)] = jnp.zeros((16,), F32)
        return 0
    lax.fori_loop(0, _CHUNK, zrow, 0)
    rbase = sid * _ROWS_PER_TILE
    for k in range(13):
        pltpu.sync_copy(a0, acc.at[pl.ds(rbase + k * _CHUNK, _CHUNK)])
    pltpu.sync_copy(a0.at[pl.ds(0, 8)], acc.at[pl.ds(rbase + 13 * _CHUNK, 8)])
    plsc.subcore_barrier()

    args = (acc, pk, (s0, s1, s2), (d0, d1, d2), (a0, a1, a2), (b0, b1, b2),
            (g0, g1, g2), (e0, e1, e2), (sc0, sc1, sc2), sid)

    @pl.when(cid == 0)
    def _lo():
        _sc_core(mlo, elo, out_lo, pk1d, *args)

    @pl.when(cid == 1)
    def _hi():
        _sc_core(mhi, ehi, out_hi, pk1d, *args)


@functools.cache
def _sc_edge_kernel():
    return pl.kernel(
        _sc_edge_body,
        out_type=[jax.ShapeDtypeStruct((N_PAD, HALF), F32),
                  jax.ShapeDtypeStruct((N_PAD, HALF), F32)],
        mesh=plsc.VectorSubcoreMesh(core_axis_name="c", subcore_axis_name="s"),
        scratch_types=[
            pltpu.VMEM((_PER_TILE * _CHUNK,), jnp.int32),
            pltpu.VMEM((_CHUNK,), jnp.int32),
            pltpu.VMEM((_CHUNK,), jnp.int32),
            pltpu.VMEM((_CHUNK,), jnp.int32),
            pltpu.VMEM((_CHUNK,), jnp.int32),
            pltpu.VMEM((_CHUNK,), jnp.int32),
            pltpu.VMEM((_CHUNK,), jnp.int32),
            pltpu.VMEM((_CHUNK, HALF), F32),
            pltpu.VMEM((_CHUNK, HALF), F32),
            pltpu.VMEM((_CHUNK, HALF), F32),
            pltpu.VMEM((_CHUNK, HALF), F32),
            pltpu.VMEM((_CHUNK, HALF), F32),
            pltpu.VMEM((_CHUNK, HALF), F32),
            pltpu.VMEM_SHARED((N_PAD, HALF), F32),
            pltpu.SemaphoreType.DMA,
            pltpu.SemaphoreType.DMA,
            pltpu.SemaphoreType.DMA,
            pltpu.SemaphoreType.DMA,
            pltpu.SemaphoreType.DMA,
            pltpu.SemaphoreType.DMA,
            pltpu.SemaphoreType.DMA,
            pltpu.SemaphoreType.DMA,
            pltpu.SemaphoreType.DMA,
        ],
    )


def _edge_phase(msg_lo, msg_hi, ea_lo, ea_hi, pk1d):
    return _sc_edge_kernel()(msg_lo, msg_hi, ea_lo, ea_hi, pk1d)


def kernel(x, edge_index, edge_attr, batch, y, w_n2l, w_e2l, p_node_conv,
           trans_node_1, trans_node_2, h1_weight, h2_weight):
    src = edge_index[0].astype(jnp.int32)
    dst = edge_index[1].astype(jnp.int32)
    x8 = jnp.pad(x, ((0, 0), (0, 8 - x.shape[1])))
    w8 = jnp.pad(w_n2l, ((0, 8 - w_n2l.shape[0]), (0, 0)))
    ea8 = jnp.pad(edge_attr, ((0, E_PAD - E), (0, 8 - edge_attr.shape[1])))
    we8 = jnp.pad(w_e2l, ((0, 8 - w_e2l.shape[0]), (0, 0)))
    t1_lo = trans_node_1[:HALF]
    t1_hi = trans_node_1[HALF:]
    h1_top = h1_weight[:EMBED]
    h1_bot = h1_weight[EMBED:]
    batch2d = batch.astype(jnp.int32).reshape(1, N)
    y2d = y.astype(jnp.int32).reshape(B, 1)
    pk1d = jnp.pad((dst << 16) | src, (0, E_PAD - E),
                   constant_values=(N << 16))

    h, msg_lo, msg_hi = _node_prologue(x8, w8, p_node_conv)
    ea_lo, ea_hi = _edge_prologue(ea8, we8)

    for _ in range(T):
        e2n_lo, e2n_hi = _edge_phase(msg_lo, msg_hi, ea_lo, ea_hi, pk1d)
        h, msg_lo, msg_hi = _update(e2n_lo, e2n_hi, h, t1_lo, t1_hi, t2=trans_node_2,
                                    p=p_node_conv)
    return _epilogue(h, batch2d, y2d, h1_top, h1_bot, h2_weight)


# speedup vs baseline: 1.3337x; 1.3337x over previous
import functools

import jax
import jax.numpy as jnp
from jax import lax
from jax.experimental import pallas as pl
from jax.experimental.pallas import tpu as pltpu
from jax.experimental.pallas import tpu_sc as plsc

N = 10000
E = 160000
B = 64
EMBED = 256
HALF = 128
T = 4

F32 = jnp.float32


def _dot(a, b, precision=None):
    return lax.dot_general(a, b, (((1,), (0,)), ((), ())),
                           preferred_element_type=F32, precision=precision)


def _node_prologue_body(x_ref, w_ref, p_ref, h_ref, mlo_ref, mhi_ref):
    hb = jax.nn.relu(_dot(x_ref[...], w_ref[...]))
    h_ref[...] = hb
    m = _dot(hb, p_ref[...])
    mlo_ref[...] = m[:, :HALF]
    mhi_ref[...] = m[:, HALF:]


def _node_prologue(x8, w8, p):
    blk = 1000
    return pl.pallas_call(
        _node_prologue_body,
        grid=(N // blk,),
        in_specs=[
            pl.BlockSpec((blk, 8), lambda r: (r, 0)),
            pl.BlockSpec((8, EMBED), lambda r: (0, 0)),
            pl.BlockSpec((EMBED, EMBED), lambda r: (0, 0)),
        ],
        out_specs=[
            pl.BlockSpec((blk, EMBED), lambda r: (r, 0)),
            pl.BlockSpec((blk, HALF), lambda r: (r, 0)),
            pl.BlockSpec((blk, HALF), lambda r: (r, 0)),
        ],
        out_shape=[
            jax.ShapeDtypeStruct((N, EMBED), F32),
            jax.ShapeDtypeStruct((N, HALF), F32),
            jax.ShapeDtypeStruct((N, HALF), F32),
        ],
    )(x8, w8, p)


def _edge_prologue_body(ea_ref, w_ref, lo_ref, hi_ref):
    e = _dot(ea_ref[...], w_ref[...])
    lo_ref[...] = e[:, :HALF]
    hi_ref[...] = e[:, HALF:]


def _edge_prologue(ea8, we8):
    blk = 4096
    rows = ea8.shape[0]
    return pl.pallas_call(
        _edge_prologue_body,
        grid=(rows // blk,),
        in_specs=[
            pl.BlockSpec((blk, 8), lambda r: (r, 0)),
            pl.BlockSpec((8, EMBED), lambda r: (0, 0)),
        ],
        out_specs=[
            pl.BlockSpec((blk, HALF), lambda r: (r, 0)),
            pl.BlockSpec((blk, HALF), lambda r: (r, 0)),
        ],
        out_shape=[
            jax.ShapeDtypeStruct((rows, HALF), F32),
            jax.ShapeDtypeStruct((rows, HALF), F32),
        ],
    )(ea8, we8)


def _update_body(elo_ref, ehi_ref, h_ref, t1lo_ref, t1hi_ref, t2_ref, p_ref,
                 h2_ref, mlo_ref, mhi_ref):
    acc = _dot(elo_ref[...], t1lo_ref[...])
    acc = acc + _dot(ehi_ref[...], t1hi_ref[...])
    acc = acc + _dot(h_ref[...], t2_ref[...])
    hb = jax.nn.relu(acc)
    h2_ref[...] = hb
    m = _dot(hb, p_ref[...])
    mlo_ref[...] = m[:, :HALF]
    mhi_ref[...] = m[:, HALF:]


def _update(e2n_lo, e2n_hi, h, t1_lo, t1_hi, t2, p):
    blk = 1000
    return pl.pallas_call(
        _update_body,
        grid=(N // blk,),
        in_specs=[
            pl.BlockSpec((blk, HALF), lambda r: (r, 0)),
            pl.BlockSpec((blk, HALF), lambda r: (r, 0)),
            pl.BlockSpec((blk, EMBED), lambda r: (r, 0)),
            pl.BlockSpec((HALF, EMBED), lambda r: (0, 0)),
            pl.BlockSpec((HALF, EMBED), lambda r: (0, 0)),
            pl.BlockSpec((EMBED, EMBED), lambda r: (0, 0)),
            pl.BlockSpec((EMBED, EMBED), lambda r: (0, 0)),
        ],
        out_specs=[
            pl.BlockSpec((blk, EMBED), lambda r: (r, 0)),
            pl.BlockSpec((blk, HALF), lambda r: (r, 0)),
            pl.BlockSpec((blk, HALF), lambda r: (r, 0)),
        ],
        out_shape=[
            jax.ShapeDtypeStruct((N, EMBED), F32),
            jax.ShapeDtypeStruct((N, HALF), F32),
            jax.ShapeDtypeStruct((N, HALF), F32),
        ],
    )(e2n_lo, e2n_hi, h, t1_lo, t1_hi, t2, p)


def _epilogue_body(h_ref, batch_ref, y_ref, h1t_ref, h1b_ref, h2_ref, o_ref):
    hv = h_ref[...]
    bsel = jnp.broadcast_to(batch_ref[...], (B, N)) == lax.broadcasted_iota(
        jnp.int32, (B, N), 0)
    bo = jnp.where(bsel, 1.0, 0.0).astype(F32)
    y_pot = _dot(bo, hv, precision=lax.Precision.HIGHEST)
    ysel = lax.broadcasted_iota(jnp.int32, (B, N), 1) == jnp.broadcast_to(
        y_ref[...], (B, N))
    yo = jnp.where(ysel, 1.0, 0.0).astype(F32)
    act = _dot(yo, hv, precision=lax.Precision.HIGHEST)
    hid = jax.nn.relu(_dot(act, h1t_ref[...]) + _dot(y_pot, h1b_ref[...]))
    o_ref[...] = _dot(hid, h2_ref[...])


def _epilogue(h, batch2d, y2d, h1_top, h1_bot, h2):
    return pl.pallas_call(
        _epilogue_body,
        out_shape=jax.ShapeDtypeStruct((B, 1), F32),
    )(h, batch2d, y2d, h1_top, h1_bot, h2)


_CHUNK = 64
_PER_TILE = 160
_NSUB = 16
_CHUNKS = _PER_TILE * _NSUB
E_PAD = _CHUNKS * _CHUNK
N_PAD = 10112
_ROWS_PER_TILE = N_PAD // _NSUB


def _sc_core(msg, ea, out, pk1d, acc, pk, src_st, dst_st, ab, bb,
             gs, es, ss, sid):
    base = sid * _PER_TILE
    nidx = _PER_TILE * _CHUNK

    pltpu.sync_copy(pk1d.at[pl.ds(sid * nidx, nidx)], pk)

    def stage(j, b):
        for k in range(_CHUNK // 16):
            pv = pk[pl.ds(j * _CHUNK + k * 16, 16)]
            src_st[b][pl.ds(k * 16, 16)] = pv & 0xFFFF
            dst_st[b][pl.ds(k * 16, 16)] = lax.shift_right_logical(pv, 16)

    def issue_in(j, b):
        pltpu.async_copy(msg.at[src_st[b]], ab[b], gs[b])
        pltpu.async_copy(ea.at[pl.ds((base + j) * _CHUNK, _CHUNK)], bb[b], es[b])

    def wait_in(j, b):
        pltpu.make_async_copy(msg.at[src_st[b]], ab[b], gs[b]).wait()
        pltpu.make_async_copy(ea.at[pl.ds((base + j) * _CHUNK, _CHUNK)], bb[b], es[b]).wait()

    def wait_scatter(b):
        pltpu.make_async_copy(ab[b], acc.at[dst_st[b]], ss[b]).wait()

    def compute(b):
        def row(r, _):
            for k in range(HALF // 16):
                sl = pl.ds(k * 16, 16)
                ab[b][r, sl] = jnp.maximum(ab[b][r, sl] + bb[b][r, sl], 0.0)
            return 0
        lax.fori_loop(0, _CHUNK, row, 0)

    stage(0, 0)
    stage(1, 1)
    issue_in(0, 0)
    issue_in(1, 1)

    def pair(p, _):
        for b in range(2):
            j = 2 * p + b
            wait_in(j, b)
            compute(b)
            pltpu.async_copy(ab[b], acc.at[dst_st[b]], ss[b], add=True)

            @pl.when(p < _PER_TILE // 2 - 1)
            def _refill():
                wait_scatter(b)
                stage(j + 2, b)
                issue_in(j + 2, b)
        return 0

    lax.fori_loop(0, _PER_TILE // 2, pair, 0)
    wait_scatter(0)
    wait_scatter(1)

    plsc.subcore_barrier()
    rows = pl.ds(sid * _ROWS_PER_TILE, _ROWS_PER_TILE)
    pltpu.sync_copy(acc.at[rows], out.at[rows])


def _sc_edge_body(mlo, mhi, elo, ehi, pk1d, out_lo, out_hi,
                  pk, s_st0, s_st1, d_st0, d_st1, a0, a1, b0, b1, acc,
                  g0, g1, e0, e1, s0, s1):
    cid = lax.axis_index("c")
    sid = lax.axis_index("s")

    def zrow(r, _):
        for k in range(HALF // 16):
            a0[r, pl.ds(k * 16, 16)] = jnp.zeros((16,), F32)
        return 0
    lax.fori_loop(0, _CHUNK, zrow, 0)
    rbase = sid * _ROWS_PER_TILE
    for k in range(9):
        pltpu.sync_copy(a0, acc.at[pl.ds(rbase + k * _CHUNK, _CHUNK)])
    pltpu.sync_copy(a0.at[pl.ds(0, 56)], acc.at[pl.ds(rbase + 9 * _CHUNK, 56)])
    plsc.subcore_barrier()

    args = (acc, pk, (s_st0, s_st1), (d_st0, d_st1), (a0, a1), (b0, b1),
            (g0, g1), (e0, e1), (s0, s1), sid)

    @pl.when(cid == 0)
    def _lo():
        _sc_core(mlo, elo, out_lo, pk1d, *args)

    @pl.when(cid == 1)
    def _hi():
        _sc_core(mhi, ehi, out_hi, pk1d, *args)


@functools.cache
def _sc_edge_kernel():
    return pl.kernel(
        _sc_edge_body,
        out_type=[jax.ShapeDtypeStruct((N_PAD, HALF), F32),
                  jax.ShapeDtypeStruct((N_PAD, HALF), F32)],
        mesh=plsc.VectorSubcoreMesh(core_axis_name="c", subcore_axis_name="s"),
        scratch_types=[
            pltpu.VMEM((_PER_TILE * _CHUNK,), jnp.int32),
            pltpu.VMEM((_CHUNK,), jnp.int32),
            pltpu.VMEM((_CHUNK,), jnp.int32),
            pltpu.VMEM((_CHUNK,), jnp.int32),
            pltpu.VMEM((_CHUNK,), jnp.int32),
            pltpu.VMEM((_CHUNK, HALF), F32),
            pltpu.VMEM((_CHUNK, HALF), F32),
            pltpu.VMEM((_CHUNK, HALF), F32),
            pltpu.VMEM((_CHUNK, HALF), F32),
            pltpu.VMEM_SHARED((N_PAD, HALF), F32),
            pltpu.SemaphoreType.DMA,
            pltpu.SemaphoreType.DMA,
            pltpu.SemaphoreType.DMA,
            pltpu.SemaphoreType.DMA,
            pltpu.SemaphoreType.DMA,
            pltpu.SemaphoreType.DMA,
        ],
    )


def _edge_phase(msg_lo, msg_hi, ea_lo, ea_hi, pk1d):
    return _sc_edge_kernel()(msg_lo, msg_hi, ea_lo, ea_hi, pk1d)


def kernel(x, edge_index, edge_attr, batch, y, w_n2l, w_e2l, p_node_conv,
           trans_node_1, trans_node_2, h1_weight, h2_weight):
    src = edge_index[0].astype(jnp.int32)
    dst = edge_index[1].astype(jnp.int32)
    x8 = jnp.pad(x, ((0, 0), (0, 8 - x.shape[1])))
    w8 = jnp.pad(w_n2l, ((0, 8 - w_n2l.shape[0]), (0, 0)))
    ea8 = jnp.pad(edge_attr, ((0, E_PAD - E), (0, 8 - edge_attr.shape[1])))
    we8 = jnp.pad(w_e2l, ((0, 8 - w_e2l.shape[0]), (0, 0)))
    t1_lo = trans_node_1[:HALF]
    t1_hi = trans_node_1[HALF:]
    h1_top = h1_weight[:EMBED]
    h1_bot = h1_weight[EMBED:]
    batch2d = batch.astype(jnp.int32).reshape(1, N)
    y2d = y.astype(jnp.int32).reshape(B, 1)
    pk1d = jnp.pad((dst << 16) | src, (0, E_PAD - E),
                   constant_values=(N << 16))

    h, msg_lo, msg_hi = _node_prologue(x8, w8, p_node_conv)
    ea_lo, ea_hi = _edge_prologue(ea8, we8)

    for _ in range(T):
        e2n_lo, e2n_hi = _edge_phase(msg_lo, msg_hi, ea_lo, ea_hi, pk1d)
        h, msg_lo, msg_hi = _update(e2n_lo, e2n_hi, h, t1_lo, t1_hi, t2=trans_node_2,
                                    p=p_node_conv)
    return _epilogue(h, batch2d, y2d, h1_top, h1_bot, h2_weight)


# speedup vs baseline: 1.3720x; 1.0287x over previous
import functools

import jax
import jax.numpy as jnp
from jax import lax
from jax.experimental import pallas as pl
from jax.experimental.pallas import tpu as pltpu
from jax.experimental.pallas import tpu_sc as plsc

N = 10000
E = 160000
B = 64
EMBED = 256
HALF = 128
T = 4

F32 = jnp.float32


def _dot(a, b, precision=None):
    return lax.dot_general(a, b, (((1,), (0,)), ((), ())),
                           preferred_element_type=F32, precision=precision)


def _node_prologue_body(x_ref, w_ref, p_ref, h_ref, mlo_ref, mhi_ref):
    hb = jax.nn.relu(_dot(x_ref[...], w_ref[...]))
    h_ref[...] = hb
    m = _dot(hb, p_ref[...])
    mlo_ref[...] = m[:, :HALF]
    mhi_ref[...] = m[:, HALF:]


def _node_prologue(x8, w8, p):
    blk = 1000
    return pl.pallas_call(
        _node_prologue_body,
        grid=(N // blk,),
        in_specs=[
            pl.BlockSpec((blk, 8), lambda r: (r, 0)),
            pl.BlockSpec((8, EMBED), lambda r: (0, 0)),
            pl.BlockSpec((EMBED, EMBED), lambda r: (0, 0)),
        ],
        out_specs=[
            pl.BlockSpec((blk, EMBED), lambda r: (r, 0)),
            pl.BlockSpec((blk, HALF), lambda r: (r, 0)),
            pl.BlockSpec((blk, HALF), lambda r: (r, 0)),
        ],
        out_shape=[
            jax.ShapeDtypeStruct((N, EMBED), F32),
            jax.ShapeDtypeStruct((N, HALF), F32),
            jax.ShapeDtypeStruct((N, HALF), F32),
        ],
    )(x8, w8, p)


def _edge_prologue_body(ea_ref, w_ref, lo_ref, hi_ref):
    e = _dot(ea_ref[...], w_ref[...])
    lo_ref[...] = e[:, :HALF]
    hi_ref[...] = e[:, HALF:]


def _edge_prologue(ea8, we8):
    blk = 4096
    rows = ea8.shape[0]
    return pl.pallas_call(
        _edge_prologue_body,
        grid=(rows // blk,),
        in_specs=[
            pl.BlockSpec((blk, 8), lambda r: (r, 0)),
            pl.BlockSpec((8, EMBED), lambda r: (0, 0)),
        ],
        out_specs=[
            pl.BlockSpec((blk, HALF), lambda r: (r, 0)),
            pl.BlockSpec((blk, HALF), lambda r: (r, 0)),
        ],
        out_shape=[
            jax.ShapeDtypeStruct((rows, HALF), F32),
            jax.ShapeDtypeStruct((rows, HALF), F32),
        ],
    )(ea8, we8)


def _update_body(elo_ref, ehi_ref, h_ref, t1lo_ref, t1hi_ref, t2_ref, p_ref,
                 h2_ref, mlo_ref, mhi_ref):
    acc = _dot(elo_ref[...], t1lo_ref[...])
    acc = acc + _dot(ehi_ref[...], t1hi_ref[...])
    acc = acc + _dot(h_ref[...], t2_ref[...])
    hb = jax.nn.relu(acc)
    h2_ref[...] = hb
    m = _dot(hb, p_ref[...])
    mlo_ref[...] = m[:, :HALF]
    mhi_ref[...] = m[:, HALF:]


def _update(e2n_lo, e2n_hi, h, t1_lo, t1_hi, t2, p):
    blk = 1000
    return pl.pallas_call(
        _update_body,
        grid=(N // blk,),
        in_specs=[
            pl.BlockSpec((blk, HALF), lambda r: (r, 0)),
            pl.BlockSpec((blk, HALF), lambda r: (r, 0)),
            pl.BlockSpec((blk, EMBED), lambda r: (r, 0)),
            pl.BlockSpec((HALF, EMBED), lambda r: (0, 0)),
            pl.BlockSpec((HALF, EMBED), lambda r: (0, 0)),
            pl.BlockSpec((EMBED, EMBED), lambda r: (0, 0)),
            pl.BlockSpec((EMBED, EMBED), lambda r: (0, 0)),
        ],
        out_specs=[
            pl.BlockSpec((blk, EMBED), lambda r: (r, 0)),
            pl.BlockSpec((blk, HALF), lambda r: (r, 0)),
            pl.BlockSpec((blk, HALF), lambda r: (r, 0)),
        ],
        out_shape=[
            jax.ShapeDtypeStruct((N, EMBED), F32),
            jax.ShapeDtypeStruct((N, HALF), F32),
            jax.ShapeDtypeStruct((N, HALF), F32),
        ],
    )(e2n_lo, e2n_hi, h, t1_lo, t1_hi, t2, p)


def _epilogue_body(h_ref, batch_ref, y_ref, h1t_ref, h1b_ref, h2_ref, o_ref):
    hv = h_ref[...]
    bsel = jnp.broadcast_to(batch_ref[...], (B, N)) == lax.broadcasted_iota(
        jnp.int32, (B, N), 0)
    bo = jnp.where(bsel, 1.0, 0.0).astype(F32)
    y_pot = _dot(bo, hv, precision=lax.Precision.HIGHEST)
    ysel = lax.broadcasted_iota(jnp.int32, (B, N), 1) == jnp.broadcast_to(
        y_ref[...], (B, N))
    yo = jnp.where(ysel, 1.0, 0.0).astype(F32)
    act = _dot(yo, hv, precision=lax.Precision.HIGHEST)
    hid = jax.nn.relu(_dot(act, h1t_ref[...]) + _dot(y_pot, h1b_ref[...]))
    o_ref[...] = _dot(hid, h2_ref[...])


def _epilogue(h, batch2d, y2d, h1_top, h1_bot, h2):
    return pl.pallas_call(
        _epilogue_body,
        out_shape=jax.ShapeDtypeStruct((B, 1), F32),
    )(h, batch2d, y2d, h1_top, h1_bot, h2)


_CHUNK = 64
_PER_TILE = 160
_NSUB = 16
_CHUNKS = _PER_TILE * _NSUB
E_PAD = _CHUNKS * _CHUNK
N_PAD = 10112
_ROWS_PER_TILE = N_PAD // _NSUB


def _sc_core(msg, ea, out, pk1d, acc, pk, src_st, dst_st, ab, bb,
             gs, es, ss, sid):
    base = sid * _PER_TILE
    nidx = _PER_TILE * _CHUNK

    pltpu.sync_copy(pk1d.at[pl.ds(sid * nidx, nidx)], pk)

    def stage_src(j, b):
        for k in range(_CHUNK // 16):
            pv = pk[pl.ds(j * _CHUNK + k * 16, 16)]
            src_st[b][pl.ds(k * 16, 16)] = pv & 0xFFFF

    def stage_dst(j, b):
        for k in range(_CHUNK // 16):
            pv = pk[pl.ds(j * _CHUNK + k * 16, 16)]
            dst_st[b][pl.ds(k * 16, 16)] = lax.shift_right_logical(pv, 16)

    def issue_gather(b):
        pltpu.async_copy(msg.at[src_st[b]], ab[b], gs[b])

    def issue_ea(j, b):
        pltpu.async_copy(ea.at[pl.ds((base + j) * _CHUNK, _CHUNK)], bb[b], es[b])

    def wait_in(j, b):
        pltpu.make_async_copy(msg.at[src_st[b]], ab[b], gs[b]).wait()
        pltpu.make_async_copy(ea.at[pl.ds((base + j) * _CHUNK, _CHUNK)], bb[b], es[b]).wait()

    def wait_scatter(b):
        pltpu.make_async_copy(bb[b], acc.at[dst_st[b]], ss[b]).wait()

    def compute(b):
        def row(r, _):
            for k in range(HALF // 16):
                sl = pl.ds(k * 16, 16)
                bb[b][r, sl] = jnp.maximum(ab[b][r, sl] + bb[b][r, sl], 0.0)
            return 0
        lax.fori_loop(0, _CHUNK, row, 0)

    for b in (0, 1):
        stage_src(b, b)
        stage_dst(b, b)
        issue_gather(b)
        issue_ea(b, b)

    def pair(p, _):
        for b in range(2):
            j = 2 * p + b
            wait_in(j, b)
            compute(b)
            pltpu.async_copy(bb[b], acc.at[dst_st[b]], ss[b], add=True)

            @pl.when(p < _PER_TILE // 2 - 1)
            def _refill():
                stage_src(j + 2, b)
                issue_gather(b)
                wait_scatter(b)
                stage_dst(j + 2, b)
                issue_ea(j + 2, b)
        return 0

    lax.fori_loop(0, _PER_TILE // 2, pair, 0)
    wait_scatter(0)
    wait_scatter(1)

    plsc.subcore_barrier()
    rows = pl.ds(sid * _ROWS_PER_TILE, _ROWS_PER_TILE)
    pltpu.sync_copy(acc.at[rows], out.at[rows])


def _sc_edge_body(mlo, mhi, elo, ehi, pk1d, out_lo, out_hi,
                  pk, s_st0, s_st1, d_st0, d_st1, a0, a1, b0, b1, acc,
                  g0, g1, e0, e1, s0, s1):
    cid = lax.axis_index("c")
    sid = lax.axis_index("s")

    def zrow(r, _):
        for k in range(HALF // 16):
            a0[r, pl.ds(k * 16, 16)] = jnp.zeros((16,), F32)
        return 0
    lax.fori_loop(0, _CHUNK, zrow, 0)
    rbase = sid * _ROWS_PER_TILE
    for k in range(9):
        pltpu.sync_copy(a0, acc.at[pl.ds(rbase + k * _CHUNK, _CHUNK)])
    pltpu.sync_copy(a0.at[pl.ds(0, 56)], acc.at[pl.ds(rbase + 9 * _CHUNK, 56)])
    plsc.subcore_barrier()

    args = (acc, pk, (s_st0, s_st1), (d_st0, d_st1), (a0, a1), (b0, b1),
            (g0, g1), (e0, e1), (s0, s1), sid)

    @pl.when(cid == 0)
    def _lo():
        _sc_core(mlo, elo, out_lo, pk1d, *args)

    @pl.when(cid == 1)
    def _hi():
        _sc_core(mhi, ehi, out_hi, pk1d, *args)


@functools.cache
def _sc_edge_kernel():
    return pl.kernel(
        _sc_edge_body,
        out_type=[jax.ShapeDtypeStruct((N_PAD, HALF), F32),
                  jax.ShapeDtypeStruct((N_PAD, HALF), F32)],
        mesh=plsc.VectorSubcoreMesh(core_axis_name="c", subcore_axis_name="s"),
        scratch_types=[
            pltpu.VMEM((_PER_TILE * _CHUNK,), jnp.int32),
            pltpu.VMEM((_CHUNK,), jnp.int32),
            pltpu.VMEM((_CHUNK,), jnp.int32),
            pltpu.VMEM((_CHUNK,), jnp.int32),
            pltpu.VMEM((_CHUNK,), jnp.int32),
            pltpu.VMEM((_CHUNK, HALF), F32),
            pltpu.VMEM((_CHUNK, HALF), F32),
            pltpu.VMEM((_CHUNK, HALF), F32),
            pltpu.VMEM((_CHUNK, HALF), F32),
            pltpu.VMEM_SHARED((N_PAD, HALF), F32),
            pltpu.SemaphoreType.DMA,
            pltpu.SemaphoreType.DMA,
            pltpu.SemaphoreType.DMA,
            pltpu.SemaphoreType.DMA,
            pltpu.SemaphoreType.DMA,
            pltpu.SemaphoreType.DMA,
        ],
    )


def _edge_phase(msg_lo, msg_hi, ea_lo, ea_hi, pk1d):
    return _sc_edge_kernel()(msg_lo, msg_hi, ea_lo, ea_hi, pk1d)


def kernel(x, edge_index, edge_attr, batch, y, w_n2l, w_e2l, p_node_conv,
           trans_node_1, trans_node_2, h1_weight, h2_weight):
    src = edge_index[0].astype(jnp.int32)
    dst = edge_index[1].astype(jnp.int32)
    x8 = jnp.pad(x, ((0, 0), (0, 8 - x.shape[1])))
    w8 = jnp.pad(w_n2l, ((0, 8 - w_n2l.shape[0]), (0, 0)))
    ea8 = jnp.pad(edge_attr, ((0, E_PAD - E), (0, 8 - edge_attr.shape[1])))
    we8 = jnp.pad(w_e2l, ((0, 8 - w_e2l.shape[0]), (0, 0)))
    t1_lo = trans_node_1[:HALF]
    t1_hi = trans_node_1[HALF:]
    h1_top = h1_weight[:EMBED]
    h1_bot = h1_weight[EMBED:]
    batch2d = batch.astype(jnp.int32).reshape(1, N)
    y2d = y.astype(jnp.int32).reshape(B, 1)
    pk1d = jnp.pad((dst << 16) | src, (0, E_PAD - E),
                   constant_values=(N << 16))

    h, msg_lo, msg_hi = _node_prologue(x8, w8, p_node_conv)
    ea_lo, ea_hi = _edge_prologue(ea8, we8)

    for _ in range(T):
        e2n_lo, e2n_hi = _edge_phase(msg_lo, msg_hi, ea_lo, ea_hi, pk1d)
        h, msg_lo, msg_hi = _update(e2n_lo, e2n_hi, h, t1_lo, t1_hi, t2=trans_node_2,
                                    p=p_node_conv)
    return _epilogue(h, batch2d, y2d, h1_top, h1_bot, h2_weight)
